# Initial kernel scaffold; baseline (speedup 1.0000x reference)
import jax
import jax.numpy as jnp
from jax.experimental import pallas as pl


def kernel(seq, table):
    raise NotImplementedError("write your pallas kernel here")


# speedup vs baseline: 1.5119x; 1.5119x over previous
import jax
import jax.numpy as jnp
from jax.experimental import pallas as pl

_BS = 512


def _add_kernel(seq_ref, table_ref, out_ref):
    out_ref[...] = seq_ref[...] + table_ref[...]


def kernel(seq, table):
    batch, seq_len, dim = seq.shape
    bs = _BS
    grid = (seq_len // bs, batch)
    return pl.pallas_call(
        _add_kernel,
        grid=grid,
        in_specs=[
            pl.BlockSpec((1, bs, dim), lambda s, b: (b, s, 0)),
            pl.BlockSpec((bs, dim), lambda s, b: (s, 0)),
        ],
        out_specs=pl.BlockSpec((1, bs, dim), lambda s, b: (b, s, 0)),
        out_shape=jax.ShapeDtypeStruct(seq.shape, seq.dtype),
    )(seq, table)


# speedup vs baseline: 1.7323x; 1.1458x over previous
import jax
import jax.numpy as jnp
from jax.experimental import pallas as pl

_BS = 512


def _add_kernel(seq_ref, table_ref, out_ref):
    out_ref[...] = seq_ref[...] + table_ref[...]


def kernel(seq, table):
    batch, seq_len, dim = seq.shape
    bs = _BS
    grid = (seq_len // bs,)
    return pl.pallas_call(
        _add_kernel,
        grid=grid,
        in_specs=[
            pl.BlockSpec((batch, bs, dim), lambda s: (0, s, 0)),
            pl.BlockSpec((bs, dim), lambda s: (s, 0)),
        ],
        out_specs=pl.BlockSpec((batch, bs, dim), lambda s: (0, s, 0)),
        out_shape=jax.ShapeDtypeStruct(seq.shape, seq.dtype),
    )(seq, table)


# speedup vs baseline: 1.7388x; 1.0038x over previous
import jax
import jax.numpy as jnp
from jax.experimental import pallas as pl

_BS = 2048


def _add_kernel(seq_ref, table_ref, out_ref):
    out_ref[...] = seq_ref[...] + table_ref[...]


def kernel(seq, table):
    batch, seq_len, dim = seq.shape
    bs = _BS
    flat = seq.reshape(batch * seq_len, dim)
    t_blocks = seq_len // bs
    out = pl.pallas_call(
        _add_kernel,
        grid=(t_blocks, batch),
        in_specs=[
            pl.BlockSpec((bs, dim), lambda t, b: (b * t_blocks + t, 0)),
            pl.BlockSpec((bs, dim), lambda t, b: (t, 0)),
        ],
        out_specs=pl.BlockSpec((bs, dim), lambda t, b: (b * t_blocks + t, 0)),
        out_shape=jax.ShapeDtypeStruct(flat.shape, flat.dtype),
    )(flat, table)
    return out.reshape(seq.shape)


# speedup vs baseline: 1.7400x; 1.0007x over previous
import jax
import jax.numpy as jnp
from jax.experimental import pallas as pl
from jax.experimental.pallas import tpu as pltpu

_BS = 2048


def _add_kernel(seq_ref, table_ref, out_ref):
    out_ref[...] = seq_ref[...] + table_ref[...]


def kernel(seq, table):
    batch, seq_len, dim = seq.shape
    bs = _BS
    flat = seq.reshape(batch * seq_len, dim)
    t_blocks = seq_len // bs
    out = pl.pallas_call(
        _add_kernel,
        grid=(t_blocks, batch),
        in_specs=[
            pl.BlockSpec((bs, dim), lambda t, b: (b * t_blocks + t, 0)),
            pl.BlockSpec((bs, dim), lambda t, b: (t, 0)),
        ],
        out_specs=pl.BlockSpec((bs, dim), lambda t, b: (b * t_blocks + t, 0)),
        out_shape=jax.ShapeDtypeStruct(flat.shape, flat.dtype),
        compiler_params=pltpu.CompilerParams(
            vmem_limit_bytes=112 * 1024 * 1024,
        ),
    )(flat, table)
    return out.reshape(seq.shape)
